# baseline (speedup 1.0000x reference)
import jax
import jax.numpy as jnp
from jax import lax
from jax.experimental import pallas as pl
from jax.experimental.pallas import tpu as pltpu
from jax.experimental.pallas import tpu_sc as plsc

_B = 16384
_L = 50
_D = 128
_TOT = _B * _L
_NC = 2
_NS = 16
_NW = _NC * _NS
_PER_W = _TOT // _NW
_CHUNK = 128
_NCH = _PER_W // _CHUNK

_MM_BLK = 2000


def _proj_body(x_ref, w_ref, o_ref):
    o_ref[...] = lax.dot_general(
        x_ref[...], w_ref[...],
        (((1,), (1,)), ((), ())),
        preferred_element_type=jnp.float32,
    )


def _project_table(emb, W):
    m = emb.shape[0]
    grid = m // _MM_BLK
    return pl.pallas_call(
        _proj_body,
        grid=(grid,),
        in_specs=[
            pl.BlockSpec((_MM_BLK, _D), lambda i: (i, 0)),
            pl.BlockSpec((_D, _D), lambda i: (0, 0)),
        ],
        out_specs=pl.BlockSpec((_MM_BLK, _D), lambda i: (i, 0)),
        out_shape=jax.ShapeDtypeStruct((m, _D), jnp.float32),
    )(emb, W)


_LP = 56
_TOTP = _B * _LP
_ROWS_W = _B // _NW
_GCHUNK = 2 * _LP
_STEP_ROWS = 4
_OCHUNK = _STEP_ROWS * _LP
_NB = _ROWS_W // _STEP_ROWS
_IDX_W = _ROWS_W * _LP


def _gather_body(tab_hbm, idx_hbm, out_hbm, idx_v, rows_a, rows_b,
                 gsem_a, gsem_b, wsem_a, wsem_b):
    wid = lax.axis_index("s") * _NC + lax.axis_index("c")
    pltpu.sync_copy(idx_hbm.at[pl.ds(wid * _IDX_W, _IDX_W)], idx_v)
    out_base = wid * _IDX_W

    def fire(step, rows, gsem):
        off = step * _OCHUNK
        pltpu.async_copy(
            tab_hbm.at[idx_v.at[pl.ds(off, _GCHUNK)]],
            rows.at[pl.ds(0, _GCHUNK)], gsem)
        pltpu.async_copy(
            tab_hbm.at[idx_v.at[pl.ds(off + _GCHUNK, _GCHUNK)]],
            rows.at[pl.ds(_GCHUNK, _GCHUNK)], gsem)

    def drain_gathers(rows, gsem):
        pltpu.make_async_copy(
            tab_hbm.at[idx_v.at[pl.ds(0, _GCHUNK)]],
            rows.at[pl.ds(0, _GCHUNK)], gsem).wait()
        pltpu.make_async_copy(
            tab_hbm.at[idx_v.at[pl.ds(0, _GCHUNK)]],
            rows.at[pl.ds(_GCHUNK, _GCHUNK)], gsem).wait()

    def wait_write(rows, wsem):
        pltpu.make_async_copy(rows, out_hbm.at[pl.ds(0, _OCHUNK)], wsem).wait()

    fire(0, rows_a, gsem_a)

    def step(j, carry):
        even = (j % 2) == 0

        @pl.when(jnp.logical_and(j + 1 < _NB, even))
        def _():
            @pl.when(j >= 1)
            def _():
                wait_write(rows_b, wsem_b)
            fire(j + 1, rows_b, gsem_b)

        @pl.when(jnp.logical_and(j + 1 < _NB, jnp.logical_not(even)))
        def _():
            wait_write(rows_a, wsem_a)
            fire(j + 1, rows_a, gsem_a)

        @pl.when(even)
        def _():
            drain_gathers(rows_a, gsem_a)
            pltpu.async_copy(
                rows_a, out_hbm.at[pl.ds(out_base + j * _OCHUNK, _OCHUNK)],
                wsem_a)

        @pl.when(jnp.logical_not(even))
        def _():
            drain_gathers(rows_b, gsem_b)
            pltpu.async_copy(
                rows_b, out_hbm.at[pl.ds(out_base + j * _OCHUNK, _OCHUNK)],
                wsem_b)

        return carry

    lax.fori_loop(0, _NB, step, 0)
    wait_write(rows_a, wsem_a)
    wait_write(rows_b, wsem_b)


_gather = pl.kernel(
    _gather_body,
    out_type=jax.ShapeDtypeStruct((_TOTP, _D), jnp.float32),
    mesh=plsc.VectorSubcoreMesh(
        core_axis_name="c", subcore_axis_name="s",
        num_cores=_NC, num_subcores=_NS,
    ),
    scratch_types=[
        pltpu.VMEM((_IDX_W,), jnp.int32),
        pltpu.VMEM((_OCHUNK, _D), jnp.float32),
        pltpu.VMEM((_OCHUNK, _D), jnp.float32),
        pltpu.SemaphoreType.DMA,
        pltpu.SemaphoreType.DMA,
        pltpu.SemaphoreType.DMA,
        pltpu.SemaphoreType.DMA,
    ],
)


@jax.jit
def kernel(idx, emb, W):
    proj = _project_table(emb, W)
    idx_flat = jnp.pad(idx.astype(jnp.int32).reshape(_B, _L),
                       ((0, 0), (0, _LP - _L))).reshape(-1)
    out = _gather(proj, idx_flat)
    return out.reshape(_B, _LP, _D)[:, :_L, :]


# speedup vs baseline: 7.9573x; 7.9573x over previous
import jax
import jax.numpy as jnp
from jax import lax
from jax.experimental import pallas as pl
from jax.experimental.pallas import tpu as pltpu
from jax.experimental.pallas import tpu_sc as plsc

_B = 16384
_L = 50
_D = 128
_TOT = _B * _L
_NC = 2
_NS = 16
_NW = _NC * _NS
_PER_W = _TOT // _NW
_CHUNK = 128
_NCH = _PER_W // _CHUNK

_MM_BLK = 2000


def _proj_body(x_ref, w_ref, o_ref):
    o_ref[...] = lax.dot_general(
        x_ref[...], w_ref[...],
        (((1,), (1,)), ((), ())),
        preferred_element_type=jnp.float32,
    )


def _project_table(emb, W):
    m = emb.shape[0]
    grid = m // _MM_BLK
    return pl.pallas_call(
        _proj_body,
        grid=(grid,),
        in_specs=[
            pl.BlockSpec((_MM_BLK, _D), lambda i: (i, 0)),
            pl.BlockSpec((_D, _D), lambda i: (0, 0)),
        ],
        out_specs=pl.BlockSpec((_MM_BLK, _D), lambda i: (i, 0)),
        out_shape=jax.ShapeDtypeStruct((m, _D), jnp.float32),
    )(emb, W)


_ROWS_W = _B // _NW
_STEP_ROWS = 4
_NB = _ROWS_W // _STEP_ROWS


def _gather_body(tab_hbm, idx_hbm, out_hbm, idx_v, rows_a, rows_b,
                 sem_a, sem_b):
    wid = lax.axis_index("s") * _NC + lax.axis_index("c")
    row0 = wid * _ROWS_W
    pltpu.sync_copy(idx_hbm.at[pl.ds(row0, _ROWS_W)], idx_v)

    def fire(step, rows, sem):
        for k in range(_STEP_ROWS):
            pltpu.async_copy(
                tab_hbm.at[idx_v.at[step * _STEP_ROWS + k]],
                rows.at[k], sem)

    def drain(rows, sem):
        for k in range(_STEP_ROWS):
            pltpu.make_async_copy(
                tab_hbm.at[idx_v.at[0]], rows.at[k], sem).wait()

    fire(0, rows_a, sem_a)

    def step(j, carry):
        even = (j % 2) == 0

        @pl.when(jnp.logical_and(j + 1 < _NB, even))
        def _():
            fire(j + 1, rows_b, sem_b)

        @pl.when(jnp.logical_and(j + 1 < _NB, jnp.logical_not(even)))
        def _():
            fire(j + 1, rows_a, sem_a)

        @pl.when(even)
        def _():
            drain(rows_a, sem_a)
            pltpu.sync_copy(
                rows_a, out_hbm.at[pl.ds(row0 + j * _STEP_ROWS, _STEP_ROWS)])

        @pl.when(jnp.logical_not(even))
        def _():
            drain(rows_b, sem_b)
            pltpu.sync_copy(
                rows_b, out_hbm.at[pl.ds(row0 + j * _STEP_ROWS, _STEP_ROWS)])

        return carry

    lax.fori_loop(0, _NB, step, 0)


_gather = pl.kernel(
    _gather_body,
    out_type=jax.ShapeDtypeStruct((_B, _L, _D), jnp.float32),
    mesh=plsc.VectorSubcoreMesh(
        core_axis_name="c", subcore_axis_name="s",
        num_cores=_NC, num_subcores=_NS,
    ),
    scratch_types=[
        pltpu.VMEM((_ROWS_W, _L), jnp.int32),
        pltpu.VMEM((_STEP_ROWS, _L, _D), jnp.float32),
        pltpu.VMEM((_STEP_ROWS, _L, _D), jnp.float32),
        pltpu.SemaphoreType.DMA,
        pltpu.SemaphoreType.DMA,
    ],
    compiler_params=pltpu.CompilerParams(use_tc_tiling_on_sc=True),
)


@jax.jit
def kernel(idx, emb, W):
    proj = _project_table(emb, W)
    return _gather(proj, idx.astype(jnp.int32))


# speedup vs baseline: 13.9317x; 1.7508x over previous
import jax
import jax.numpy as jnp
from jax import lax
from jax.experimental import pallas as pl
from jax.experimental.pallas import tpu as pltpu
from jax.experimental.pallas import tpu_sc as plsc

_B = 16384
_L = 50
_D = 128
_TOT = _B * _L
_NC = 2
_NS = 16
_NW = _NC * _NS
_PER_W = _TOT // _NW
_CHUNK = 128
_NCH = _PER_W // _CHUNK

_MM_BLK = 2000


def _proj_body(x_ref, w_ref, o_ref):
    o_ref[...] = lax.dot_general(
        x_ref[...], w_ref[...],
        (((1,), (1,)), ((), ())),
        preferred_element_type=jnp.float32,
    )


def _project_table(emb, W):
    m = emb.shape[0]
    grid = m // _MM_BLK
    return pl.pallas_call(
        _proj_body,
        grid=(grid,),
        in_specs=[
            pl.BlockSpec((_MM_BLK, _D), lambda i: (i, 0)),
            pl.BlockSpec((_D, _D), lambda i: (0, 0)),
        ],
        out_specs=pl.BlockSpec((_MM_BLK, _D), lambda i: (i, 0)),
        out_shape=jax.ShapeDtypeStruct((m, _D), jnp.float32),
    )(emb, W)


_PER_W = _TOT // _NW
_CHUNK = 128
_NCH = _PER_W // _CHUNK


def _gather_body(tab_hbm, idx_hbm, out_hbm, idx_v, rows_a, rows_b,
                 gsem_a, gsem_b, wsem_a, wsem_b):
    wid = lax.axis_index("s") * _NC + lax.axis_index("c")
    pltpu.sync_copy(idx_hbm.at[pl.ds(wid * _NCH, _NCH)], idx_v)
    out_base = wid * _PER_W

    fire = lambda j, rows, gsem: pltpu.async_copy(
        tab_hbm.at[idx_v.at[j]], rows, gsem)
    drain = lambda rows, gsem: pltpu.make_async_copy(
        tab_hbm.at[idx_v.at[0]], rows, gsem).wait()
    store = lambda j, rows, wsem: pltpu.async_copy(
        rows, out_hbm.at[pl.ds(out_base + j * _CHUNK, _CHUNK)], wsem)
    drain_store = lambda rows, wsem: pltpu.make_async_copy(
        rows, out_hbm.at[pl.ds(0, _CHUNK)], wsem).wait()

    fire(0, rows_a, gsem_a)

    def step(j, carry):
        even = (j % 2) == 0

        @pl.when(jnp.logical_and(j + 1 < _NCH, even))
        def _():
            @pl.when(j >= 1)
            def _():
                drain_store(rows_b, wsem_b)
            fire(j + 1, rows_b, gsem_b)

        @pl.when(jnp.logical_and(j + 1 < _NCH, jnp.logical_not(even)))
        def _():
            drain_store(rows_a, wsem_a)
            fire(j + 1, rows_a, gsem_a)

        @pl.when(even)
        def _():
            drain(rows_a, gsem_a)
            store(j, rows_a, wsem_a)

        @pl.when(jnp.logical_not(even))
        def _():
            drain(rows_b, gsem_b)
            store(j, rows_b, wsem_b)

        return carry

    lax.fori_loop(0, _NCH, step, 0)
    drain_store(rows_a, wsem_a)
    drain_store(rows_b, wsem_b)


_gather = pl.kernel(
    _gather_body,
    out_type=jax.ShapeDtypeStruct((_TOT, _D), jnp.float32),
    mesh=plsc.VectorSubcoreMesh(
        core_axis_name="c", subcore_axis_name="s",
        num_cores=_NC, num_subcores=_NS,
    ),
    scratch_types=[
        pltpu.VMEM((_NCH, _CHUNK), jnp.int32),
        pltpu.VMEM((_CHUNK, _D), jnp.float32),
        pltpu.VMEM((_CHUNK, _D), jnp.float32),
        pltpu.SemaphoreType.DMA,
        pltpu.SemaphoreType.DMA,
        pltpu.SemaphoreType.DMA,
        pltpu.SemaphoreType.DMA,
    ],
)


@jax.jit
def kernel(idx, emb, W):
    proj = _project_table(emb, W)
    idx_t = idx.astype(jnp.int32).T.reshape(_NW * _NCH, _CHUNK)
    flat = _gather(proj, idx_t)
    return flat.reshape(_L, _B, _D).transpose(1, 0, 2)


# speedup vs baseline: 14.5449x; 1.0440x over previous
import jax
import jax.numpy as jnp
from jax import lax
from jax.experimental import pallas as pl
from jax.experimental.pallas import tpu as pltpu
from jax.experimental.pallas import tpu_sc as plsc

_B = 16384
_L = 50
_D = 128
_TOT = _B * _L
_NC = 2
_NS = 16
_NW = _NC * _NS
_PER_W = _TOT // _NW
_CHUNK = 128
_NCH = _PER_W // _CHUNK

_MM_BLK = 5000


def _proj_body(x_ref, w_ref, o_ref):
    o_ref[...] = lax.dot_general(
        x_ref[...], w_ref[...],
        (((1,), (1,)), ((), ())),
        preferred_element_type=jnp.float32,
    )


def _project_table(emb, W):
    m = emb.shape[0]
    grid = m // _MM_BLK
    return pl.pallas_call(
        _proj_body,
        grid=(grid,),
        in_specs=[
            pl.BlockSpec((_MM_BLK, _D), lambda i: (i, 0)),
            pl.BlockSpec((_D, _D), lambda i: (0, 0)),
        ],
        out_specs=pl.BlockSpec((_MM_BLK, _D), lambda i: (i, 0)),
        out_shape=jax.ShapeDtypeStruct((m, _D), jnp.float32),
    )(emb, W)


_PER_W = _TOT // _NW
_IROWS = 1
_CHUNK = _IROWS * 128
_NIR = _PER_W // 128
_NCH = _PER_W // _CHUNK


def _gather_body(tab_hbm, idx_hbm, out_hbm, idx_v, rows_a, rows_b,
                 gsem_a, gsem_b, wsem_a, wsem_b):
    wid = lax.axis_index("s") * _NC + lax.axis_index("c")
    pltpu.sync_copy(idx_hbm.at[pl.ds(wid * _NIR, _NIR)], idx_v)
    out_base = wid * _PER_W

    fire = lambda j, rows, gsem: pltpu.async_copy(
        tab_hbm.at[idx_v.at[j]], rows, gsem)
    drain = lambda rows, gsem: pltpu.make_async_copy(
        tab_hbm.at[idx_v.at[0]], rows, gsem).wait()
    store = lambda j, rows, wsem: pltpu.async_copy(
        rows, out_hbm.at[pl.ds(out_base + j * _CHUNK, _CHUNK)], wsem)
    drain_store = lambda rows, wsem: pltpu.make_async_copy(
        rows, out_hbm.at[pl.ds(0, _CHUNK)], wsem).wait()

    fire(0, rows_a, gsem_a)

    def step(j, carry):
        even = (j % 2) == 0

        @pl.when(jnp.logical_and(j + 1 < _NCH, even))
        def _():
            @pl.when(j >= 1)
            def _():
                drain_store(rows_b, wsem_b)
            fire(j + 1, rows_b, gsem_b)

        @pl.when(jnp.logical_and(j + 1 < _NCH, jnp.logical_not(even)))
        def _():
            drain_store(rows_a, wsem_a)
            fire(j + 1, rows_a, gsem_a)

        @pl.when(even)
        def _():
            drain(rows_a, gsem_a)
            store(j, rows_a, wsem_a)

        @pl.when(jnp.logical_not(even))
        def _():
            drain(rows_b, gsem_b)
            store(j, rows_b, wsem_b)

        return carry

    lax.fori_loop(0, _NCH, step, 0)
    drain_store(rows_a, wsem_a)
    drain_store(rows_b, wsem_b)


_gather = pl.kernel(
    _gather_body,
    out_type=jax.ShapeDtypeStruct((_TOT, _D), jnp.float32),
    mesh=plsc.VectorSubcoreMesh(
        core_axis_name="c", subcore_axis_name="s",
        num_cores=_NC, num_subcores=_NS,
    ),
    scratch_types=[
        pltpu.VMEM((_NIR, 128), jnp.int32),
        pltpu.VMEM((_CHUNK, _D), jnp.float32),
        pltpu.VMEM((_CHUNK, _D), jnp.float32),
        pltpu.SemaphoreType.DMA,
        pltpu.SemaphoreType.DMA,
        pltpu.SemaphoreType.DMA,
        pltpu.SemaphoreType.DMA,
    ],
)


@jax.jit
def kernel(idx, emb, W):
    proj = _project_table(emb, W)
    idx_t = idx.astype(jnp.int32).T.reshape(_NW * _NIR, 128)
    flat = _gather(proj, idx_t)
    return flat.reshape(_L, _B, _D).transpose(1, 0, 2)
